# Initial kernel scaffold; baseline (speedup 1.0000x reference)
import jax
import jax.numpy as jnp
from jax.experimental import pallas as pl


def kernel(h, edge_index, node2depth, Wz_w, Wz_b, Uz_w, Uz_b, Wr_w, Wr_b, Ur_w, Ur_b, Wh_w, Wh_b, Uh_w, Uh_b):
    raise NotImplementedError("write your pallas kernel here")


# speedup vs baseline: 102.5180x; 102.5180x over previous
import functools

import jax
import jax.numpy as jnp
from jax import lax
from jax.experimental import pallas as pl
from jax.experimental.pallas import tpu as pltpu
from jax.experimental.pallas import tpu_sc as plsc

N = 100000
D = 10
DP = 16
E = 6400000
NC = 2
NS = 16
NW = NC * NS
NROWS = 100352
SUB = 128
NSUB = 8
CHUNK = SUB * NSUB
STEPS = 196
EPT = CHUNK * STEPS
E_PAD = EPT * NW
ER = E_PAD // SUB
BN = 2048
GRID = NROWS // BN


def _sc_aggregate_body(hm_hbm, src_hbm, dst_hbm, zeros_hbm, out_hbm,
                       acc, dstbuf, srcbuf, rows, gsem):
    c = lax.axis_index("c")
    s = lax.axis_index("s")
    wid = c * NS + s
    rpt = NROWS // NS
    pltpu.sync_copy(zeros_hbm.at[pl.ds(s * rpt, rpt)], acc.at[pl.ds(s * rpt, rpt)])
    plsc.subcore_barrier()

    row0 = wid * (EPT // SUB)

    def step(ci, carry):
        rbase = row0 + ci * NSUB
        pltpu.sync_copy(dst_hbm.at[pl.ds(rbase, NSUB)], dstbuf)
        pltpu.sync_copy(src_hbm.at[pl.ds(rbase, NSUB)], srcbuf)
        cps = []
        for j in range(NSUB):
            cps.append(pltpu.async_copy(
                hm_hbm.at[dstbuf.at[j]], rows.at[pl.ds(j * SUB, SUB)], gsem))
        for cp in cps:
            cp.wait()
        for j in range(NSUB):
            pltpu.sync_copy(rows.at[pl.ds(j * SUB, SUB)],
                            acc.at[srcbuf.at[j]], add=True)
        return carry

    lax.fori_loop(0, STEPS, step, 0)
    plsc.subcore_barrier()
    pltpu.sync_copy(acc.at[pl.ds(s * rpt, rpt)], out_hbm.at[c, pl.ds(s * rpt, rpt)])


@functools.partial(jax.jit, donate_argnums=())
def _sc_aggregate(hm, src2d, dst2d, zeros):
    mesh = plsc.VectorSubcoreMesh(core_axis_name="c", subcore_axis_name="s")
    return pl.kernel(
        _sc_aggregate_body,
        out_type=jax.ShapeDtypeStruct((NC, NROWS, DP), jnp.float32),
        mesh=mesh,
        scratch_types=[
            pltpu.VMEM_SHARED((NROWS, DP), jnp.float32),
            pltpu.VMEM((NSUB, SUB), jnp.int32),
            pltpu.VMEM((NSUB, SUB), jnp.int32),
            pltpu.VMEM((CHUNK, DP), jnp.float32),
            pltpu.SemaphoreType.DMA,
        ],
        compiler_params=pltpu.CompilerParams(use_tc_tiling_on_sc=False),
    )(hm, src2d, dst2d, zeros)


def _gru_math(x, h, WT, B):
    dot = functools.partial(jnp.dot, preferred_element_type=jnp.float32,
                            precision=lax.Precision.HIGHEST)
    z = jax.nn.sigmoid(dot(x, WT[0]) + dot(h, WT[1]) + B[0:1, :])
    r = jax.nn.sigmoid(dot(x, WT[2]) + dot(h, WT[3]) + B[1:2, :])
    hh = jnp.tanh(dot(x, WT[4]) + dot(r * h, WT[5]) + B[2:3, :])
    return z * h + (1.0 - z) * hh


def _tc_gru0_body(x0_ref, x1_ref, h_ref, act1_ref, WT_ref, B_ref,
                  hout_ref, hm1_ref):
    x = x0_ref[...] + x1_ref[...]
    hn = _gru_math(x, h_ref[...], WT_ref[...], B_ref[...])
    hout_ref[...] = hn
    hm1_ref[...] = hn * act1_ref[...]


def _tc_gru1_body(x0_ref, x1_ref, h_ref, act1_ref, WT_ref, B_ref, hout_ref):
    a = act1_ref[...]
    x = (x0_ref[...] + x1_ref[...]) * a
    hn = _gru_math(x, h_ref[...], WT_ref[...], B_ref[...])
    hout_ref[...] = jnp.where(a > 0.0, hn, h_ref[...])


def _node_spec():
    return pl.BlockSpec((BN, DP), lambda i: (i, 0))


def _full_specs():
    return [
        pl.BlockSpec((BN, 1), lambda i: (i, 0)),
        pl.BlockSpec((6, DP, DP), lambda i: (0, 0, 0)),
        pl.BlockSpec((8, DP), lambda i: (0, 0)),
    ]


@jax.jit
def _tc_gru0(x0, x1, h, act1, WT, B):
    out = jax.ShapeDtypeStruct((NROWS, DP), jnp.float32)
    return pl.pallas_call(
        _tc_gru0_body,
        grid=(GRID,),
        in_specs=[_node_spec(), _node_spec(), _node_spec()] + _full_specs(),
        out_specs=[_node_spec(), _node_spec()],
        out_shape=[out, out],
    )(x0, x1, h, act1, WT, B)


@jax.jit
def _tc_gru1(x0, x1, h, act1, WT, B):
    out = jax.ShapeDtypeStruct((NROWS, DP), jnp.float32)
    return pl.pallas_call(
        _tc_gru1_body,
        grid=(GRID,),
        in_specs=[_node_spec(), _node_spec(), _node_spec()] + _full_specs(),
        out_specs=_node_spec(),
        out_shape=out,
    )(x0, x1, h, act1, WT, B)


def _pad_w(w):
    return jnp.zeros((DP, DP), jnp.float32).at[:D, :D].set(w.T)


def kernel(h, edge_index, node2depth,
           Wz_w, Wz_b, Uz_w, Uz_b,
           Wr_w, Wr_b, Ur_w, Ur_b,
           Wh_w, Wh_b, Uh_w, Uh_b):
    src = edge_index[0]
    dst = edge_index[1]
    pad = E_PAD - E
    src2d = jnp.concatenate([src, jnp.full((pad,), N, jnp.int32)]).reshape(ER, SUB)
    dst2d = jnp.concatenate([dst, jnp.full((pad,), N, jnp.int32)]).reshape(ER, SUB)

    zeros = jnp.zeros((NROWS, DP), jnp.float32)
    h0 = zeros.at[:N, :D].set(h)
    act1 = zeros[:, :1].at[:N, 0].set((node2depth <= 1).astype(jnp.float32))

    WT = jnp.stack([_pad_w(Wz_w), _pad_w(Uz_w), _pad_w(Wr_w),
                    _pad_w(Ur_w), _pad_w(Wh_w), _pad_w(Uh_w)])
    B = jnp.zeros((8, DP), jnp.float32)
    B = B.at[0, :D].set(Wz_b + Uz_b)
    B = B.at[1, :D].set(Wr_b + Ur_b)
    B = B.at[2, :D].set(Wh_b + Uh_b)

    xs0 = _sc_aggregate(h0, src2d, dst2d, zeros)
    h1, hm1 = _tc_gru0(xs0[0], xs0[1], h0, act1, WT, B)
    xs1 = _sc_aggregate(hm1, src2d, dst2d, zeros)
    h2 = _tc_gru1(xs1[0], xs1[1], h1, act1, WT, B)
    return h2[:N, :D]


# speedup vs baseline: 112.5434x; 1.0978x over previous
import functools

import jax
import jax.numpy as jnp
from jax import lax
from jax.experimental import pallas as pl
from jax.experimental.pallas import tpu as pltpu
from jax.experimental.pallas import tpu_sc as plsc

N = 100000
D = 10
DP = 16
E = 6400000
NC = 2
NS = 16
NW = NC * NS
NROWS = 100352
SUB = 128
NSUB = 6
CHUNK = SUB * NSUB
STEPS = 261
EPT = CHUNK * STEPS
E_PAD = EPT * NW
ER = E_PAD // SUB
BN = 2048
GRID = NROWS // BN


def _sc_aggregate_body(hm_hbm, src_hbm, dst_hbm, zeros_hbm, out_hbm,
                       acc, dstbuf, srcbuf, rows, gsem, ssem):
    c = lax.axis_index("c")
    s = lax.axis_index("s")
    wid = c * NS + s
    rpt = NROWS // NS
    pltpu.sync_copy(zeros_hbm.at[pl.ds(s * rpt, rpt)], acc.at[pl.ds(s * rpt, rpt)])
    plsc.subcore_barrier()

    row0 = wid * (EPT // SUB)

    def fire(ci, slot):
        rbase = row0 + ci * NSUB
        pltpu.sync_copy(dst_hbm.at[pl.ds(rbase, NSUB)], dstbuf.at[slot])
        pltpu.sync_copy(src_hbm.at[pl.ds(rbase, NSUB)], srcbuf.at[slot])
        cps = []
        for j in range(NSUB):
            cps.append(pltpu.async_copy(
                hm_hbm.at[dstbuf.at[slot, j]],
                rows.at[slot, pl.ds(j * SUB, SUB)], gsem))
        for cp in cps:
            cp.wait()
        for j in range(NSUB):
            pltpu.async_copy(rows.at[slot, pl.ds(j * SUB, SUB)],
                             acc.at[srcbuf.at[slot, j]], ssem.at[slot],
                             add=True)

    def drain(slot):
        for j in range(NSUB):
            pltpu.make_async_copy(rows.at[slot, pl.ds(j * SUB, SUB)],
                                  acc.at[srcbuf.at[slot, j]],
                                  ssem.at[slot]).wait()

    def step(ci, carry):
        slot = lax.rem(ci, 2)
        drain(slot)
        fire(ci, slot)
        return carry

    fire(0, 0)
    fire(1, 1)
    lax.fori_loop(2, STEPS, step, 0)
    drain(0)
    drain(1)
    plsc.subcore_barrier()
    pltpu.sync_copy(acc.at[pl.ds(s * rpt, rpt)], out_hbm.at[c, pl.ds(s * rpt, rpt)])


@functools.partial(jax.jit, donate_argnums=())
def _sc_aggregate(hm, src2d, dst2d, zeros):
    mesh = plsc.VectorSubcoreMesh(core_axis_name="c", subcore_axis_name="s")
    return pl.kernel(
        _sc_aggregate_body,
        out_type=jax.ShapeDtypeStruct((NC, NROWS, DP), jnp.float32),
        mesh=mesh,
        scratch_types=[
            pltpu.VMEM_SHARED((NROWS, DP), jnp.float32),
            pltpu.VMEM((2, NSUB, SUB), jnp.int32),
            pltpu.VMEM((2, NSUB, SUB), jnp.int32),
            pltpu.VMEM((2, CHUNK, DP), jnp.float32),
            pltpu.SemaphoreType.DMA,
            pltpu.SemaphoreType.DMA((2,)),
        ],
        compiler_params=pltpu.CompilerParams(use_tc_tiling_on_sc=False),
    )(hm, src2d, dst2d, zeros)


def _gru_math(x, h, WT, B):
    dot = functools.partial(jnp.dot, preferred_element_type=jnp.float32,
                            precision=lax.Precision.HIGHEST)
    z = jax.nn.sigmoid(dot(x, WT[0]) + dot(h, WT[1]) + B[0:1, :])
    r = jax.nn.sigmoid(dot(x, WT[2]) + dot(h, WT[3]) + B[1:2, :])
    hh = jnp.tanh(dot(x, WT[4]) + dot(r * h, WT[5]) + B[2:3, :])
    return z * h + (1.0 - z) * hh


def _tc_gru0_body(x0_ref, x1_ref, h_ref, act1_ref, WT_ref, B_ref,
                  hout_ref, hm1_ref):
    x = x0_ref[...] + x1_ref[...]
    hn = _gru_math(x, h_ref[...], WT_ref[...], B_ref[...])
    hout_ref[...] = hn
    hm1_ref[...] = hn * act1_ref[...]


def _tc_gru1_body(x0_ref, x1_ref, h_ref, act1_ref, WT_ref, B_ref, hout_ref):
    a = act1_ref[...]
    x = (x0_ref[...] + x1_ref[...]) * a
    hn = _gru_math(x, h_ref[...], WT_ref[...], B_ref[...])
    hout_ref[...] = jnp.where(a > 0.0, hn, h_ref[...])


def _node_spec():
    return pl.BlockSpec((BN, DP), lambda i: (i, 0))


def _full_specs():
    return [
        pl.BlockSpec((BN, 1), lambda i: (i, 0)),
        pl.BlockSpec((6, DP, DP), lambda i: (0, 0, 0)),
        pl.BlockSpec((8, DP), lambda i: (0, 0)),
    ]


@jax.jit
def _tc_gru0(x0, x1, h, act1, WT, B):
    out = jax.ShapeDtypeStruct((NROWS, DP), jnp.float32)
    return pl.pallas_call(
        _tc_gru0_body,
        grid=(GRID,),
        in_specs=[_node_spec(), _node_spec(), _node_spec()] + _full_specs(),
        out_specs=[_node_spec(), _node_spec()],
        out_shape=[out, out],
    )(x0, x1, h, act1, WT, B)


@jax.jit
def _tc_gru1(x0, x1, h, act1, WT, B):
    out = jax.ShapeDtypeStruct((NROWS, DP), jnp.float32)
    return pl.pallas_call(
        _tc_gru1_body,
        grid=(GRID,),
        in_specs=[_node_spec(), _node_spec(), _node_spec()] + _full_specs(),
        out_specs=_node_spec(),
        out_shape=out,
    )(x0, x1, h, act1, WT, B)


def _pad_w(w):
    return jnp.zeros((DP, DP), jnp.float32).at[:D, :D].set(w.T)


def kernel(h, edge_index, node2depth,
           Wz_w, Wz_b, Uz_w, Uz_b,
           Wr_w, Wr_b, Ur_w, Ur_b,
           Wh_w, Wh_b, Uh_w, Uh_b):
    src = edge_index[0]
    dst = edge_index[1]
    pad = E_PAD - E
    src2d = jnp.concatenate([src, jnp.full((pad,), N, jnp.int32)]).reshape(ER, SUB)
    dst2d = jnp.concatenate([dst, jnp.full((pad,), N, jnp.int32)]).reshape(ER, SUB)

    zeros = jnp.zeros((NROWS, DP), jnp.float32)
    h0 = zeros.at[:N, :D].set(h)
    act1 = zeros[:, :1].at[:N, 0].set((node2depth <= 1).astype(jnp.float32))

    WT = jnp.stack([_pad_w(Wz_w), _pad_w(Uz_w), _pad_w(Wr_w),
                    _pad_w(Ur_w), _pad_w(Wh_w), _pad_w(Uh_w)])
    B = jnp.zeros((8, DP), jnp.float32)
    B = B.at[0, :D].set(Wz_b + Uz_b)
    B = B.at[1, :D].set(Wr_b + Ur_b)
    B = B.at[2, :D].set(Wh_b + Uh_b)

    xs0 = _sc_aggregate(h0, src2d, dst2d, zeros)
    h1, hm1 = _tc_gru0(xs0[0], xs0[1], h0, act1, WT, B)
    xs1 = _sc_aggregate(hm1, src2d, dst2d, zeros)
    h2 = _tc_gru1(xs1[0], xs1[1], h1, act1, WT, B)
    return h2[:N, :D]


# speedup vs baseline: 242.0562x; 2.1508x over previous
import functools

import jax
import jax.numpy as jnp
from jax import lax
from jax.experimental import pallas as pl
from jax.experimental.pallas import tpu as pltpu
from jax.experimental.pallas import tpu_sc as plsc

N = 100000
D = 10
DP = 16
E = 6400000
NC = 2
NS = 16
NW = NC * NS
NROWS = 100352
SUB = 128
NSUB = 5
CHUNK = SUB * NSUB
NCHUNK = E // CHUNK
BASE_STEPS = NCHUNK // NW
XTRA = NCHUNK - BASE_STEPS * NW
ER = E // SUB
RING = 4
M = NROWS // 8
BM = 256
GRID = M // BM


def _sc_aggregate_body(hm_hbm, src_hbm, dst_hbm, zeros_hbm, out_hbm,
                       acc, dstbuf, srcbuf, rows, isem, gsem, ssem):
    c = lax.axis_index("c")
    s = lax.axis_index("s")
    wid = c * NS + s
    rpt = NROWS // NS
    pltpu.sync_copy(zeros_hbm.at[pl.ds(s * rpt, rpt)], acc.at[pl.ds(s * rpt, rpt)])
    plsc.subcore_barrier()

    steps = BASE_STEPS + jnp.where(wid < XTRA, 1, 0)
    chunk0 = wid * BASE_STEPS + jnp.minimum(wid, XTRA)

    def fire_idx(ci, ri):
        rbase = (chunk0 + ci) * NSUB
        pltpu.async_copy(dst_hbm.at[pl.ds(rbase, NSUB)], dstbuf.at[ri], isem.at[ri])
        pltpu.async_copy(src_hbm.at[pl.ds(rbase, NSUB)], srcbuf.at[ri], isem.at[ri])

    def wait_idx(ri):
        pltpu.make_async_copy(src_hbm.at[pl.ds(0, NSUB)], dstbuf.at[ri],
                              isem.at[ri]).wait()
        pltpu.make_async_copy(src_hbm.at[pl.ds(0, NSUB)], srcbuf.at[ri],
                              isem.at[ri]).wait()

    def drain_scatters(slot):
        for j in range(NSUB):
            pltpu.make_async_copy(rows.at[slot, pl.ds(j * SUB, SUB)],
                                  acc.at[srcbuf.at[0, j]], ssem.at[slot]).wait()

    def step(ci, carry):
        slot = lax.rem(ci, 2)
        ri = lax.rem(ci, RING)

        @pl.when(ci >= 2)
        def _():
            drain_scatters(slot)

        wait_idx(ri)

        @pl.when(ci + 2 < steps)
        def _():
            fire_idx(ci + 2, lax.rem(ci + 2, RING))

        cps = []
        for j in range(NSUB):
            cps.append(pltpu.async_copy(
                hm_hbm.at[dstbuf.at[ri, j]],
                rows.at[slot, pl.ds(j * SUB, SUB)], gsem))
        for cp in cps:
            cp.wait()
        for j in range(NSUB):
            pltpu.async_copy(rows.at[slot, pl.ds(j * SUB, SUB)],
                             acc.at[srcbuf.at[ri, j]], ssem.at[slot],
                             add=True)
        return carry

    fire_idx(0, 0)
    fire_idx(1, 1)
    lax.fori_loop(0, steps, step, 0)
    drain_scatters(0)
    drain_scatters(1)
    plsc.subcore_barrier()
    pltpu.sync_copy(acc.at[pl.ds(s * rpt, rpt)], out_hbm.at[c, pl.ds(s * rpt, rpt)])


@jax.jit
def _sc_aggregate(hm, src2d, dst2d, zeros):
    mesh = plsc.VectorSubcoreMesh(core_axis_name="c", subcore_axis_name="s")
    return pl.kernel(
        _sc_aggregate_body,
        out_type=jax.ShapeDtypeStruct((NC, NROWS, DP), jnp.float32),
        mesh=mesh,
        scratch_types=[
            pltpu.VMEM_SHARED((NROWS, DP), jnp.float32),
            pltpu.VMEM((RING, NSUB, SUB), jnp.int32),
            pltpu.VMEM((RING, NSUB, SUB), jnp.int32),
            pltpu.VMEM((2, CHUNK, DP), jnp.float32),
            pltpu.SemaphoreType.DMA((RING,)),
            pltpu.SemaphoreType.DMA,
            pltpu.SemaphoreType.DMA((2,)),
        ],
        compiler_params=pltpu.CompilerParams(use_tc_tiling_on_sc=False),
    )(hm, src2d, dst2d, zeros)


def _gru_math(x, h, W_ref, B_ref):
    dot = functools.partial(jnp.dot, preferred_element_type=jnp.float32,
                            precision=lax.Precision.HIGHEST)
    z = jax.nn.sigmoid(dot(x, W_ref[0]) + dot(h, W_ref[1]) + B_ref[0:1, :])
    r = jax.nn.sigmoid(dot(x, W_ref[2]) + dot(h, W_ref[3]) + B_ref[1:2, :])
    hh = jnp.tanh(dot(x, W_ref[4]) + dot(r * h, W_ref[5]) + B_ref[2:3, :])
    return z * h + (1.0 - z) * hh


def _tc_gru0_body(xs_ref, h_ref, act_ref, W_ref, B_ref, hout_ref, hm1_ref):
    x = xs_ref[0] + xs_ref[1]
    h = h_ref[...]
    hn = _gru_math(x, h, W_ref, B_ref)
    hout_ref[...] = hn
    hm1_ref[...] = hn * act_ref[...]


def _tc_gru1_body(xs_ref, h_ref, act_ref, W_ref, B_ref, hout_ref):
    a = act_ref[...]
    x = (xs_ref[0] + xs_ref[1]) * a
    h = h_ref[...]
    hn = _gru_math(x, h, W_ref, B_ref)
    hout_ref[...] = jnp.where(a > 0.0, hn, h)


def _tc_specs():
    blk = pl.BlockSpec((BM, 128), lambda i: (i, 0))
    return [
        pl.BlockSpec((2, BM, 128), lambda i: (0, i, 0)),
        blk,
        blk,
        pl.BlockSpec((6, 128, 128), lambda i: (0, 0, 0)),
        pl.BlockSpec((8, 128), lambda i: (0, 0)),
    ], blk


@jax.jit
def _tc_gru0(xs, h, actp, Wbd, Bt):
    specs, blk = _tc_specs()
    out = jax.ShapeDtypeStruct((M, 128), jnp.float32)
    return pl.pallas_call(
        _tc_gru0_body,
        grid=(GRID,),
        in_specs=specs,
        out_specs=[blk, blk],
        out_shape=[out, out],
    )(xs, h, actp, Wbd, Bt)


@jax.jit
def _tc_gru1(xs, h, actp, Wbd, Bt):
    specs, blk = _tc_specs()
    out = jax.ShapeDtypeStruct((M, 128), jnp.float32)
    return pl.pallas_call(
        _tc_gru1_body,
        grid=(GRID,),
        in_specs=specs,
        out_specs=blk,
        out_shape=out,
    )(xs, h, actp, Wbd, Bt)


def _pad_w(w):
    w16 = jnp.zeros((DP, DP), jnp.float32).at[:D, :D].set(w.T)
    return jnp.kron(jnp.eye(8, dtype=jnp.float32), w16)


def _pad_b(b):
    return jnp.tile(jnp.zeros((DP,), jnp.float32).at[:D].set(b), 8)


def kernel(h, edge_index, node2depth,
           Wz_w, Wz_b, Uz_w, Uz_b,
           Wr_w, Wr_b, Ur_w, Ur_b,
           Wh_w, Wh_b, Uh_w, Uh_b):
    src2d = edge_index[0].reshape(ER, SUB)
    dst2d = edge_index[1].reshape(ER, SUB)

    zeros = jnp.zeros((NROWS, DP), jnp.float32)
    h0 = zeros.at[:N, :D].set(h)
    act1 = jnp.zeros((NROWS,), jnp.float32).at[:N].set(
        (node2depth <= 1).astype(jnp.float32))
    actp = jnp.broadcast_to(act1[:, None], (NROWS, DP)).reshape(M, 128)

    Wbd = jnp.stack([_pad_w(Wz_w), _pad_w(Uz_w), _pad_w(Wr_w),
                     _pad_w(Ur_w), _pad_w(Wh_w), _pad_w(Uh_w)])
    Bt = jnp.zeros((8, 128), jnp.float32)
    Bt = Bt.at[0].set(_pad_b(Wz_b + Uz_b))
    Bt = Bt.at[1].set(_pad_b(Wr_b + Ur_b))
    Bt = Bt.at[2].set(_pad_b(Wh_b + Uh_b))

    xs0 = _sc_aggregate(h0, src2d, dst2d, zeros)
    h1, hm1 = _tc_gru0(xs0.reshape(NC, M, 128), h0.reshape(M, 128), actp, Wbd, Bt)
    xs1 = _sc_aggregate(hm1.reshape(NROWS, DP), src2d, dst2d, zeros)
    h2 = _tc_gru1(xs1.reshape(NC, M, 128), h1, actp, Wbd, Bt)
    return h2.reshape(NROWS, DP)[:N, :D]


# speedup vs baseline: 248.9342x; 1.0284x over previous
import functools

import jax
import jax.numpy as jnp
from jax import lax
from jax.experimental import pallas as pl
from jax.experimental.pallas import tpu as pltpu
from jax.experimental.pallas import tpu_sc as plsc

N = 100000
D = 10
DP = 16
E = 6400000
NC = 2
NS = 16
NW = NC * NS
NROWS = 100352
SUB = 128
NSUB = 5
CHUNK = SUB * NSUB
NCHUNK = E // CHUNK
BASE_STEPS = NCHUNK // NW
XTRA = NCHUNK - BASE_STEPS * NW
ER = E // SUB
RING = 4
M = NROWS // 8
BM = 256
GRID = M // BM


def _sc_aggregate_body(hm_hbm, edges_hbm, zeros_hbm, out_hbm,
                       acc, idxbuf, rows, isem, gsem, ssem):
    c = lax.axis_index("c")
    s = lax.axis_index("s")
    wid = c * NS + s
    rpt = NROWS // NS
    pltpu.sync_copy(zeros_hbm.at[pl.ds(s * rpt, rpt)], acc.at[pl.ds(s * rpt, rpt)])
    plsc.subcore_barrier()

    steps = BASE_STEPS + jnp.where(wid < XTRA, 1, 0)
    chunk0 = wid * BASE_STEPS + jnp.minimum(wid, XTRA)

    def fire_idx(ci, ri):
        ebase = (chunk0 + ci) * CHUNK
        pltpu.async_copy(edges_hbm.at[:, pl.ds(ebase, CHUNK)], idxbuf.at[ri],
                         isem.at[ri])

    def wait_idx(ri):
        pltpu.make_async_copy(edges_hbm.at[:, pl.ds(0, CHUNK)], idxbuf.at[ri],
                              isem.at[ri]).wait()

    def drain_scatter(slot):
        pltpu.make_async_copy(rows.at[slot], acc.at[idxbuf.at[0, 0]],
                              ssem.at[slot]).wait()

    def step(ci, carry):
        slot = lax.rem(ci, 2)
        ri = lax.rem(ci, RING)

        @pl.when(ci >= 2)
        def _():
            drain_scatter(slot)

        wait_idx(ri)

        @pl.when(ci + 2 < steps)
        def _():
            fire_idx(ci + 2, lax.rem(ci + 2, RING))

        pltpu.async_copy(hm_hbm.at[idxbuf.at[ri, 1]], rows.at[slot], gsem).wait()
        pltpu.async_copy(rows.at[slot], acc.at[idxbuf.at[ri, 0]], ssem.at[slot],
                         add=True)
        return carry

    fire_idx(0, 0)
    fire_idx(1, 1)
    lax.fori_loop(0, steps, step, 0)
    drain_scatter(0)
    drain_scatter(1)
    plsc.subcore_barrier()
    pltpu.sync_copy(acc.at[pl.ds(s * rpt, rpt)], out_hbm.at[c, pl.ds(s * rpt, rpt)])


@jax.jit
def _sc_aggregate(hm, edges, zeros):
    mesh = plsc.VectorSubcoreMesh(core_axis_name="c", subcore_axis_name="s")
    return pl.kernel(
        _sc_aggregate_body,
        out_type=jax.ShapeDtypeStruct((NC, NROWS, DP), jnp.float32),
        mesh=mesh,
        scratch_types=[
            pltpu.VMEM_SHARED((NROWS, DP), jnp.float32),
            pltpu.VMEM((RING, 2, CHUNK), jnp.int32),
            pltpu.VMEM((2, CHUNK, DP), jnp.float32),
            pltpu.SemaphoreType.DMA((RING,)),
            pltpu.SemaphoreType.DMA,
            pltpu.SemaphoreType.DMA((2,)),
        ],
        compiler_params=pltpu.CompilerParams(use_tc_tiling_on_sc=False),
    )(hm, edges, zeros)


def _gru_math(x, h, W_ref, B_ref):
    dot = functools.partial(jnp.dot, preferred_element_type=jnp.float32,
                            precision=lax.Precision.HIGHEST)
    z = jax.nn.sigmoid(dot(x, W_ref[0]) + dot(h, W_ref[1]) + B_ref[0:1, :])
    r = jax.nn.sigmoid(dot(x, W_ref[2]) + dot(h, W_ref[3]) + B_ref[1:2, :])
    hh = jnp.tanh(dot(x, W_ref[4]) + dot(r * h, W_ref[5]) + B_ref[2:3, :])
    return z * h + (1.0 - z) * hh


def _tc_gru0_body(xs_ref, h_ref, act_ref, W_ref, B_ref, hout_ref, hm1_ref):
    x = xs_ref[0] + xs_ref[1]
    h = h_ref[...]
    hn = _gru_math(x, h, W_ref, B_ref)
    hout_ref[...] = hn
    hm1_ref[...] = hn * act_ref[...]


def _tc_gru1_body(xs_ref, h_ref, act_ref, W_ref, B_ref, hout_ref):
    a = act_ref[...]
    x = (xs_ref[0] + xs_ref[1]) * a
    h = h_ref[...]
    hn = _gru_math(x, h, W_ref, B_ref)
    hout_ref[...] = jnp.where(a > 0.0, hn, h)


def _tc_specs():
    blk = pl.BlockSpec((BM, 128), lambda i: (i, 0))
    return [
        pl.BlockSpec((2, BM, 128), lambda i: (0, i, 0)),
        blk,
        blk,
        pl.BlockSpec((6, 128, 128), lambda i: (0, 0, 0)),
        pl.BlockSpec((8, 128), lambda i: (0, 0)),
    ], blk


@jax.jit
def _tc_gru0(xs, h, actp, Wbd, Bt):
    specs, blk = _tc_specs()
    out = jax.ShapeDtypeStruct((M, 128), jnp.float32)
    return pl.pallas_call(
        _tc_gru0_body,
        grid=(GRID,),
        in_specs=specs,
        out_specs=[blk, blk],
        out_shape=[out, out],
    )(xs, h, actp, Wbd, Bt)


@jax.jit
def _tc_gru1(xs, h, actp, Wbd, Bt):
    specs, blk = _tc_specs()
    out = jax.ShapeDtypeStruct((M, 128), jnp.float32)
    return pl.pallas_call(
        _tc_gru1_body,
        grid=(GRID,),
        in_specs=specs,
        out_specs=blk,
        out_shape=out,
    )(xs, h, actp, Wbd, Bt)


def _pad_w(w):
    w16 = jnp.zeros((DP, DP), jnp.float32).at[:D, :D].set(w.T)
    return jnp.kron(jnp.eye(8, dtype=jnp.float32), w16)


def _pad_b(b):
    return jnp.tile(jnp.zeros((DP,), jnp.float32).at[:D].set(b), 8)


def kernel(h, edge_index, node2depth,
           Wz_w, Wz_b, Uz_w, Uz_b,
           Wr_w, Wr_b, Ur_w, Ur_b,
           Wh_w, Wh_b, Uh_w, Uh_b):
    zeros = jnp.zeros((NROWS, DP), jnp.float32)
    h0 = zeros.at[:N, :D].set(h)
    act1 = jnp.zeros((NROWS,), jnp.float32).at[:N].set(
        (node2depth <= 1).astype(jnp.float32))
    actp = jnp.broadcast_to(act1[:, None], (NROWS, DP)).reshape(M, 128)

    Wbd = jnp.stack([_pad_w(Wz_w), _pad_w(Uz_w), _pad_w(Wr_w),
                     _pad_w(Ur_w), _pad_w(Wh_w), _pad_w(Uh_w)])
    Bt = jnp.zeros((8, 128), jnp.float32)
    Bt = Bt.at[0].set(_pad_b(Wz_b + Uz_b))
    Bt = Bt.at[1].set(_pad_b(Wr_b + Ur_b))
    Bt = Bt.at[2].set(_pad_b(Wh_b + Uh_b))

    xs0 = _sc_aggregate(h0, edge_index, zeros)
    h1, hm1 = _tc_gru0(xs0.reshape(NC, M, 128), h0.reshape(M, 128), actp, Wbd, Bt)
    xs1 = _sc_aggregate(hm1.reshape(NROWS, DP), edge_index, zeros)
    h2 = _tc_gru1(xs1.reshape(NC, M, 128), h1, actp, Wbd, Bt)
    return h2.reshape(NROWS, DP)[:N, :D]
